# baseline (speedup 1.0000x reference)
import functools

import jax
import jax.numpy as jnp
from jax import lax
from jax.experimental import pallas as pl
from jax.experimental.pallas import tpu as pltpu
from jax.experimental.pallas import tpu_sc as plsc

_BATCH = 16384
_Z_DIM = 64
_NUM_WORKERS = 32
_B_PER_W = _BATCH // _NUM_WORKERS


def kernel(u, embed_mean, embed_log_var):
    idx = u.astype(jnp.int32)
    out_sds = jax.ShapeDtypeStruct((_BATCH, _Z_DIM), embed_mean.dtype)
    mesh = plsc.VectorSubcoreMesh(core_axis_name="c", subcore_axis_name="s")

    @jax.jit
    @functools.partial(
        pl.kernel,
        out_type=(out_sds, out_sds),
        mesh=mesh,
        scratch_types=[
            pltpu.VMEM((_B_PER_W,), jnp.int32),
            pltpu.VMEM((_B_PER_W, _Z_DIM), jnp.float32),
            pltpu.VMEM((_B_PER_W, _Z_DIM), jnp.float32),
            pltpu.SemaphoreType.DMA,
            pltpu.SemaphoreType.DMA,
        ],
        compiler_params=pltpu.CompilerParams(use_tc_tiling_on_sc=False),
    )
    def _gather(mean_hbm, logvar_hbm, idx_hbm, om_hbm, ov_hbm,
                idx_v, mrows_v, vrows_v, sem_m, sem_v):
        wid = lax.axis_index("s") * 2 + lax.axis_index("c")
        base = wid * _B_PER_W
        pltpu.sync_copy(idx_hbm.at[pl.ds(base, _B_PER_W)], idx_v)
        cp_m = pltpu.make_async_copy(mean_hbm.at[idx_v], mrows_v, sem_m)
        cp_v = pltpu.make_async_copy(logvar_hbm.at[idx_v], vrows_v, sem_v)
        cp_m.start()
        cp_v.start()
        cp_m.wait()
        cp_v.wait()
        pltpu.sync_copy(mrows_v, om_hbm.at[pl.ds(base, _B_PER_W)])
        pltpu.sync_copy(vrows_v, ov_hbm.at[pl.ds(base, _B_PER_W)])

    return _gather(embed_mean, embed_log_var, idx)
